# baseline (speedup 1.0000x reference)
import functools

import jax
import jax.numpy as jnp
from jax import lax
from jax.experimental import pallas as pl
from jax.experimental.pallas import tpu as pltpu
from jax.experimental.pallas import tpu_sc as plsc

_DIM = 4096
_BATCH = 16384
_NC = 2
_NS = 16
_L = 16
_NW = _NC * _NS
_RPW = _BATCH // _NW
_R = 4

_mesh = plsc.VectorSubcoreMesh(core_axis_name="c", subcore_axis_name="s")


@functools.partial(
    pl.kernel,
    mesh=_mesh,
    out_type=jax.ShapeDtypeStruct((_BATCH * _DIM,), jnp.float32),
    scratch_types=[
        pltpu.VMEM((_DIM,), jnp.int32),
        pltpu.VMEM((_R * _DIM,), jnp.float32),
        pltpu.VMEM((_R * _DIM,), jnp.float32),
    ],
    compiler_params=pltpu.CompilerParams(needs_layout_passes=False),
)
def _permute(x_hbm, perm_hbm, out_hbm, perm_v, in_v, out_v):
    wid = lax.axis_index("s") * _NC + lax.axis_index("c")
    base = wid * _RPW * _DIM
    pltpu.sync_copy(perm_hbm, perm_v)

    def chunk_body(c, carry):
        off = base + c * (_R * _DIM)
        pltpu.sync_copy(x_hbm.at[pl.ds(off, _R * _DIM)], in_v)

        def jbody(j, carry2):
            idx = perm_v[pl.ds(j * _L, _L)]
            for r in range(_R):
                out_v[pl.ds(r * _DIM + j * _L, _L)] = plsc.load_gather(
                    in_v, [idx + (r * _DIM)])
            return carry2

        lax.fori_loop(0, _DIM // _L, jbody, 0)
        pltpu.sync_copy(out_v, out_hbm.at[pl.ds(off, _R * _DIM)])
        return carry

    lax.fori_loop(0, _RPW // _R, chunk_body, 0)


def kernel(x, perm):
    z = _permute(x.reshape(-1), perm.astype(jnp.int32))
    logdet = jnp.zeros((x.shape[0],), dtype=x.dtype)
    return (z.reshape(_BATCH, _DIM), logdet)


# speedup vs baseline: 1.2169x; 1.2169x over previous
import functools

import jax
import jax.numpy as jnp
from jax import lax
from jax.experimental import pallas as pl
from jax.experimental.pallas import tpu as pltpu
from jax.experimental.pallas import tpu_sc as plsc

_DIM = 4096
_BATCH = 16384
_NC = 2
_NS = 16
_L = 16
_NW = _NC * _NS
_RPW = _BATCH // _NW
_R = 4
_RD = _R * _DIM
_NCHUNK = _RPW // _R

_mesh = plsc.VectorSubcoreMesh(core_axis_name="c", subcore_axis_name="s")


@functools.partial(
    pl.kernel,
    mesh=_mesh,
    out_type=jax.ShapeDtypeStruct((_BATCH * _DIM,), jnp.float32),
    scratch_types=[
        pltpu.VMEM((_DIM,), jnp.int32),
        pltpu.VMEM((_RD,), jnp.float32),
        pltpu.VMEM((_RD,), jnp.float32),
        pltpu.VMEM((_RD,), jnp.float32),
        pltpu.VMEM((_RD,), jnp.float32),
        pltpu.SemaphoreType.DMA((2,)),
        pltpu.SemaphoreType.DMA((2,)),
    ],
    compiler_params=pltpu.CompilerParams(needs_layout_passes=False),
)
def _permute(x_hbm, perm_hbm, out_hbm, perm_v, in0, in1, out0, out1,
             in_sem, out_sem):
    ins = (in0, in1)
    outs = (out0, out1)
    wid = lax.axis_index("s") * _NC + lax.axis_index("c")
    base = wid * _RPW * _DIM
    pltpu.sync_copy(perm_hbm, perm_v)

    pltpu.async_copy(x_hbm.at[pl.ds(base, _RD)], in0, in_sem.at[0])

    def pair_body(p, carry):
        for b in range(2):
            g = p * 2 + b

            @pl.when(g + 1 < _NCHUNK)
            def _():
                pltpu.async_copy(
                    x_hbm.at[pl.ds(base + (g + 1) * _RD, _RD)],
                    ins[1 - b], in_sem.at[1 - b])

            pltpu.make_async_copy(
                x_hbm.at[pl.ds(0, _RD)], ins[b], in_sem.at[b]).wait()

            @pl.when(g >= 2)
            def _():
                pltpu.make_async_copy(
                    outs[b], out_hbm.at[pl.ds(0, _RD)],
                    out_sem.at[b]).wait()

            in_ref = ins[b]
            out_ref = outs[b]

            def jbody(j, carry2):
                idx = perm_v[pl.ds(j * _L, _L)]
                for r in range(_R):
                    out_ref[pl.ds(r * _DIM + j * _L, _L)] = plsc.load_gather(
                        in_ref, [idx + (r * _DIM)])
                return carry2

            lax.fori_loop(0, _DIM // _L, jbody, 0)

            pltpu.async_copy(
                out_ref, out_hbm.at[pl.ds(base + g * _RD, _RD)],
                out_sem.at[b])
        return carry

    lax.fori_loop(0, _NCHUNK // 2, pair_body, 0)

    for b in range(2):
        pltpu.make_async_copy(
            outs[b], out_hbm.at[pl.ds(0, _RD)], out_sem.at[b]).wait()


def kernel(x, perm):
    z = _permute(x.reshape(-1), perm.astype(jnp.int32))
    logdet = jnp.zeros((x.shape[0],), dtype=x.dtype)
    return (z.reshape(_BATCH, _DIM), logdet)


# speedup vs baseline: 2.1608x; 1.7757x over previous
import functools

import jax
import jax.numpy as jnp
from jax import lax
from jax.experimental import pallas as pl
from jax.experimental.pallas import tpu as pltpu
from jax.experimental.pallas import tpu_sc as plsc

_DIM = 4096
_BATCH = 16384
_NC = 2
_NS = 16
_L = 16
_NW = _NC * _NS
_RPW = _BATCH // _NW
_R = 4
_RD = _R * _DIM
_NCHUNK = _RPW // _R

_mesh = plsc.VectorSubcoreMesh(core_axis_name="c", subcore_axis_name="s")


@functools.partial(
    pl.kernel,
    mesh=_mesh,
    out_type=jax.ShapeDtypeStruct((_BATCH * _DIM,), jnp.float32),
    scratch_types=[
        pltpu.VMEM((_DIM,), jnp.int32),
        pltpu.VMEM((_RD,), jnp.float32),
        pltpu.VMEM((_RD,), jnp.float32),
        pltpu.VMEM((_RD,), jnp.float32),
        pltpu.VMEM((_RD,), jnp.float32),
        pltpu.SemaphoreType.DMA((2,)),
        pltpu.SemaphoreType.DMA((2,)),
    ],
    compiler_params=pltpu.CompilerParams(needs_layout_passes=False),
)
def _permute(x_hbm, perm_hbm, out_hbm, perm_v, in0, in1, out0, out1,
             in_sem, out_sem):
    ins = (in0, in1)
    outs = (out0, out1)
    wid = lax.axis_index("s") * _NC + lax.axis_index("c")
    base = wid * _RPW * _DIM
    pltpu.sync_copy(perm_hbm, perm_v)

    pltpu.async_copy(x_hbm.at[pl.ds(base, _RD)], in0, in_sem.at[0])

    def pair_body(p, carry):
        for b in range(2):
            g = p * 2 + b

            @pl.when(g + 1 < _NCHUNK)
            def _():
                pltpu.async_copy(
                    x_hbm.at[pl.ds(base + (g + 1) * _RD, _RD)],
                    ins[1 - b], in_sem.at[1 - b])

            pltpu.make_async_copy(
                x_hbm.at[pl.ds(0, _RD)], ins[b], in_sem.at[b]).wait()

            @pl.when(g >= 2)
            def _():
                pltpu.make_async_copy(
                    outs[b], out_hbm.at[pl.ds(0, _RD)],
                    out_sem.at[b]).wait()

            in_ref = ins[b]
            out_ref = outs[b]

            @plsc.parallel_loop(0, _DIM, step=_L, unroll=4)
            def jbody(j):
                idx = perm_v[pl.ds(j, _L)]
                for r in range(_R):
                    out_ref[pl.ds(r * _DIM + j, _L)] = plsc.load_gather(
                        in_ref, [idx + (r * _DIM)])

            pltpu.async_copy(
                out_ref, out_hbm.at[pl.ds(base + g * _RD, _RD)],
                out_sem.at[b])
        return carry

    lax.fori_loop(0, _NCHUNK // 2, pair_body, 0)

    for b in range(2):
        pltpu.make_async_copy(
            outs[b], out_hbm.at[pl.ds(0, _RD)], out_sem.at[b]).wait()


def kernel(x, perm):
    z = _permute(x.reshape(-1), perm.astype(jnp.int32))
    logdet = jnp.zeros((x.shape[0],), dtype=x.dtype)
    return (z.reshape(_BATCH, _DIM), logdet)
